# Initial kernel scaffold; baseline (speedup 1.0000x reference)
import jax
import jax.numpy as jnp
from jax.experimental import pallas as pl


def kernel(input, dim, index, src):
    raise NotImplementedError("write your pallas kernel here")


# speedup vs baseline: 9.0390x; 9.0390x over previous
import functools

import jax
import jax.numpy as jnp
from jax import lax
from jax.experimental import pallas as pl
from jax.experimental.pallas import tpu as pltpu
from jax.experimental.pallas import tpu_sc as plsc

_M, _D, _B = 100000, 128, 16384
_CW = 16
_NCG = _D // _CW
_NRP = 4
_RPR = _M // _NRP
_NCH = 4
_R = _RPR // _NCH
_S = 256
_NP = _B // _S


def _body(inp_hbm, idx_hbm, src_hbm, out_hbm,
          acc, ib0, ib1, sb0, sb1, sem0, sem1):
  cid = lax.axis_index("c")
  sid = lax.axis_index("s")
  wid = sid * 2 + cid
  cg = wid % _NCG
  rp = wid // _NCG
  c0 = cg * _CW
  lanes = lax.iota(jnp.int32, _CW)

  def start(piece, ib, sb, sem):
    pltpu.async_copy(idx_hbm.at[pl.ds(piece * _S, _S), pl.ds(c0, _CW)], ib, sem)
    pltpu.async_copy(src_hbm.at[pl.ds(piece * _S, _S), pl.ds(c0, _CW)], sb, sem)

  def wait(piece, ib, sb, sem):
    pltpu.make_async_copy(
        idx_hbm.at[pl.ds(piece * _S, _S), pl.ds(c0, _CW)], ib, sem).wait()
    pltpu.make_async_copy(
        src_hbm.at[pl.ds(piece * _S, _S), pl.ds(c0, _CW)], sb, sem).wait()

  def consume(r0, ib, sb):
    def row(r, carry):
      iv = ib[r]
      sv = sb[r]
      loc = iv - r0
      msk = (loc >= 0) & (loc < _R)
      plsc.addupdate_scatter(acc, [loc, lanes], sv, mask=msk)
      return carry
    lax.fori_loop(0, _S, row, 0, unroll=4)

  def chunk(ch, carry):
    r0 = rp * _RPR + ch * _R
    pltpu.sync_copy(inp_hbm.at[pl.ds(r0, _R), pl.ds(c0, _CW)], acc)
    start(0, ib0, sb0, sem0)

    def pair(j, c2):
      pa = 2 * j
      pb = pa + 1
      start(pb, ib1, sb1, sem1)
      wait(pa, ib0, sb0, sem0)
      consume(r0, ib0, sb0)

      @pl.when(j + 1 < _NP // 2)
      def _():
        start(pa + 2, ib0, sb0, sem0)

      wait(pb, ib1, sb1, sem1)
      consume(r0, ib1, sb1)
      return c2

    lax.fori_loop(0, _NP // 2, pair, 0)
    pltpu.sync_copy(acc, out_hbm.at[pl.ds(r0, _R), pl.ds(c0, _CW)])
    return carry

  lax.fori_loop(0, _NCH, chunk, 0)


@jax.jit
def _scatter_add(inp, idx, src):
  mesh = plsc.VectorSubcoreMesh(core_axis_name="c", subcore_axis_name="s")
  run = pl.kernel(
      _body,
      out_type=jax.ShapeDtypeStruct((_M, _D), jnp.float32),
      mesh=mesh,
      compiler_params=pltpu.CompilerParams(use_tc_tiling_on_sc=False,
                           needs_layout_passes=False),
      scratch_types=[
          pltpu.VMEM((_R, _CW), jnp.float32),
          pltpu.VMEM((_S, _CW), jnp.int32),
          pltpu.VMEM((_S, _CW), jnp.int32),
          pltpu.VMEM((_S, _CW), jnp.float32),
          pltpu.VMEM((_S, _CW), jnp.float32),
          pltpu.SemaphoreType.DMA,
          pltpu.SemaphoreType.DMA,
      ],
  )
  return run(inp, idx, src)


def kernel(input, dim, index, src):
  idx = (index + dim).astype(jnp.int32)
  return _scatter_add(input, idx, src)


# speedup vs baseline: 17.0568x; 1.8870x over previous
import functools

import jax
import jax.numpy as jnp
from jax import lax
from jax.experimental import pallas as pl
from jax.experimental.pallas import tpu as pltpu
from jax.experimental.pallas import tpu_sc as plsc

_M, _D, _B = 100000, 128, 16384
_CW = 16
_NCG = _D // _CW
_NRP = 4
_RPR = _M // _NRP
_NCH = 4
_R = _RPR // _NCH
_S = 256
_NP = _B // _S


def _body(inp_hbm, idx_hbm, src_hbm, out_hbm,
          acc, ib0, ib1, sb0, sb1, sem0, sem1):
  cid = lax.axis_index("c")
  sid = lax.axis_index("s")
  wid = sid * 2 + cid
  cg = wid % _NCG
  rp = wid // _NCG
  c0 = cg * _CW
  lanes = lax.iota(jnp.int32, _CW)

  def start(piece, ib, sb, sem):
    pltpu.async_copy(idx_hbm.at[pl.ds(piece * _S, _S), pl.ds(c0, _CW)], ib, sem)
    pltpu.async_copy(src_hbm.at[pl.ds(piece * _S, _S), pl.ds(c0, _CW)], sb, sem)

  def wait(piece, ib, sb, sem):
    pltpu.make_async_copy(
        idx_hbm.at[pl.ds(piece * _S, _S), pl.ds(c0, _CW)], ib, sem).wait()
    pltpu.make_async_copy(
        src_hbm.at[pl.ds(piece * _S, _S), pl.ds(c0, _CW)], sb, sem).wait()

  def consume(r0, ib, sb):
    @plsc.parallel_loop(0, _S, unroll=8)
    def row(r):
      iv = ib[r]
      sv = sb[r]
      loc = iv - r0
      msk = (loc >= 0) & (loc < _R)
      plsc.addupdate_scatter(acc, [loc, lanes], sv, mask=msk)

  def chunk(ch, carry):
    r0 = rp * _RPR + ch * _R
    pltpu.sync_copy(inp_hbm.at[pl.ds(r0, _R), pl.ds(c0, _CW)], acc)
    start(0, ib0, sb0, sem0)

    def pair(j, c2):
      pa = 2 * j
      pb = pa + 1
      start(pb, ib1, sb1, sem1)
      wait(pa, ib0, sb0, sem0)
      consume(r0, ib0, sb0)

      @pl.when(j + 1 < _NP // 2)
      def _():
        start(pa + 2, ib0, sb0, sem0)

      wait(pb, ib1, sb1, sem1)
      consume(r0, ib1, sb1)
      return c2

    lax.fori_loop(0, _NP // 2, pair, 0)
    pltpu.sync_copy(acc, out_hbm.at[pl.ds(r0, _R), pl.ds(c0, _CW)])
    return carry

  lax.fori_loop(0, _NCH, chunk, 0)


@jax.jit
def _scatter_add(inp, idx, src):
  mesh = plsc.VectorSubcoreMesh(core_axis_name="c", subcore_axis_name="s")
  run = pl.kernel(
      _body,
      out_type=jax.ShapeDtypeStruct((_M, _D), jnp.float32),
      mesh=mesh,
      compiler_params=pltpu.CompilerParams(use_tc_tiling_on_sc=False,
                           needs_layout_passes=False),
      scratch_types=[
          pltpu.VMEM((_R, _CW), jnp.float32),
          pltpu.VMEM((_S, _CW), jnp.int32),
          pltpu.VMEM((_S, _CW), jnp.int32),
          pltpu.VMEM((_S, _CW), jnp.float32),
          pltpu.VMEM((_S, _CW), jnp.float32),
          pltpu.SemaphoreType.DMA,
          pltpu.SemaphoreType.DMA,
      ],
  )
  return run(inp, idx, src)


def kernel(input, dim, index, src):
  idx = (index + dim).astype(jnp.int32)
  return _scatter_add(input, idx, src)


# speedup vs baseline: 17.0656x; 1.0005x over previous
import functools

import jax
import jax.numpy as jnp
from jax import lax
from jax.experimental import pallas as pl
from jax.experimental.pallas import tpu as pltpu
from jax.experimental.pallas import tpu_sc as plsc

_M, _D, _B = 100000, 128, 16384
_CW = 16
_NCG = _D // _CW
_NRP = 4
_RPR = _M // _NRP
_NCH = 4
_R = _RPR // _NCH
_S = 256
_NP = _B // _S


def _body(inp_hbm, idx_hbm, src_hbm, out_hbm,
          acc, ib0, ib1, sb0, sb1, sem0, sem1):
  cid = lax.axis_index("c")
  sid = lax.axis_index("s")
  wid = sid * 2 + cid
  cg = wid % _NCG
  rp = wid // _NCG
  c0 = cg * _CW
  lanes = lax.iota(jnp.int32, _CW)

  def start(piece, ib, sb, sem):
    pltpu.async_copy(idx_hbm.at[pl.ds(piece * _S, _S), pl.ds(c0, _CW)], ib, sem)
    pltpu.async_copy(src_hbm.at[pl.ds(piece * _S, _S), pl.ds(c0, _CW)], sb, sem)

  def wait(piece, ib, sb, sem):
    pltpu.make_async_copy(
        idx_hbm.at[pl.ds(piece * _S, _S), pl.ds(c0, _CW)], ib, sem).wait()
    pltpu.make_async_copy(
        src_hbm.at[pl.ds(piece * _S, _S), pl.ds(c0, _CW)], sb, sem).wait()

  def consume(r0, ib, sb):
    @plsc.parallel_loop(0, _S, unroll=16)
    def row(r):
      iv = ib[r]
      sv = sb[r]
      loc = iv - r0
      msk = (loc >= 0) & (loc < _R)
      plsc.addupdate_scatter(acc, [loc, lanes], sv, mask=msk)

  def chunk(ch, carry):
    r0 = rp * _RPR + ch * _R
    pltpu.sync_copy(inp_hbm.at[pl.ds(r0, _R), pl.ds(c0, _CW)], acc)
    start(0, ib0, sb0, sem0)

    def pair(j, c2):
      pa = 2 * j
      pb = pa + 1
      start(pb, ib1, sb1, sem1)
      wait(pa, ib0, sb0, sem0)
      consume(r0, ib0, sb0)

      @pl.when(j + 1 < _NP // 2)
      def _():
        start(pa + 2, ib0, sb0, sem0)

      wait(pb, ib1, sb1, sem1)
      consume(r0, ib1, sb1)
      return c2

    lax.fori_loop(0, _NP // 2, pair, 0)
    pltpu.sync_copy(acc, out_hbm.at[pl.ds(r0, _R), pl.ds(c0, _CW)])
    return carry

  lax.fori_loop(0, _NCH, chunk, 0)


@jax.jit
def _scatter_add(inp, idx, src):
  mesh = plsc.VectorSubcoreMesh(core_axis_name="c", subcore_axis_name="s")
  run = pl.kernel(
      _body,
      out_type=jax.ShapeDtypeStruct((_M, _D), jnp.float32),
      mesh=mesh,
      compiler_params=pltpu.CompilerParams(use_tc_tiling_on_sc=False,
                           needs_layout_passes=False),
      scratch_types=[
          pltpu.VMEM((_R, _CW), jnp.float32),
          pltpu.VMEM((_S, _CW), jnp.int32),
          pltpu.VMEM((_S, _CW), jnp.int32),
          pltpu.VMEM((_S, _CW), jnp.float32),
          pltpu.VMEM((_S, _CW), jnp.float32),
          pltpu.SemaphoreType.DMA,
          pltpu.SemaphoreType.DMA,
      ],
  )
  return run(inp, idx, src)


def kernel(input, dim, index, src):
  idx = (index + dim).astype(jnp.int32)
  return _scatter_add(input, idx, src)
